# baseline (speedup 1.0000x reference)
import jax
import jax.numpy as jnp
from jax.experimental import pallas as pl

N = 10000
E = 320000
G = 128


def _leaky_relu(v, slope=0.2):
    return jnp.where(v >= 0, v, slope * v)


def _gat_conv(h, src, dst, W, att_src, att_dst, bias, heads, out_ch):
    Nn = h.shape[0]
    xw = (h @ W).reshape(Nn, heads, out_ch)
    a_src = (xw * att_src).sum(-1)
    a_dst = (xw * att_dst).sum(-1)
    alpha = _leaky_relu(a_src[src] + a_dst[dst])
    amax = jax.ops.segment_max(alpha, dst, num_segments=Nn)
    amax = jnp.where(jnp.isfinite(amax), amax, 0.0)
    ex = jnp.exp(alpha - amax[dst])
    denom = jax.ops.segment_sum(ex, dst, num_segments=Nn)
    coef = ex / (denom[dst] + 1e-16)
    msg = xw[src] * coef[:, :, None]
    out = jax.ops.segment_sum(msg, dst, num_segments=Nn)
    return out.reshape(Nn, heads * out_ch) + bias


def _proj_kernel(p_ref, w_ref, b_ref, o_ref):
    o_ref[...] = p_ref[...] @ w_ref[...] + b_ref[...]


def kernel(x, edge_index, node_depth, batch, type_emb, attr_emb, depth_emb,
           W1, as1, ad1, b1, W2, as2, ad2, b2, W3, as3, ad3, b3, Wp, bp):
    h = type_emb[x[:, 0]] + attr_emb[x[:, 1]] + depth_emb[node_depth.reshape(-1)]
    loop = jnp.arange(N, dtype=edge_index.dtype)
    src = jnp.concatenate([edge_index[0], loop])
    dst = jnp.concatenate([edge_index[1], loop])
    h = jax.nn.elu(_gat_conv(h, src, dst, W1, as1, ad1, b1, 4, 128))
    h = jax.nn.elu(_gat_conv(h, src, dst, W2, as2, ad2, b2, 4, 128))
    h = _gat_conv(h, src, dst, W3, as3, ad3, b3, 6, 121)
    sums = jax.ops.segment_sum(h, batch, num_segments=G)
    counts = jax.ops.segment_sum(jnp.ones((N, 1), jnp.float32), batch, num_segments=G)
    pooled = sums / jnp.maximum(counts, 1.0)
    pooled = pooled.reshape(-1, 6, 121).mean(axis=1)
    return pooled @ Wp + bp


# speedup vs baseline: 16.1235x; 16.1235x over previous
import functools

import jax
import jax.numpy as jnp
from jax import lax
from jax.experimental import pallas as pl
from jax.experimental.pallas import tpu as pltpu
from jax.experimental.pallas import tpu_sc as plsc

N = 10000
E = 320000
G = 128
NP = 10112
SENT = 10000
ET = 331776
PER_TILE = ET // 16
NCH = 162
NB = NP // 128
STRIPE = NP // 16


def _mm_first_body(h_ref, w_ref, asd_ref, xw_ref, a_ref, *, heads):
    xwb = jnp.dot(h_ref[...], w_ref[...], preferred_element_type=jnp.float32)
    a_ref[...] = jnp.dot(xwb, asd_ref[...], preferred_element_type=jnp.float32)
    for hh in range(heads):
        xw_ref[hh] = xwb[:, hh * 128:(hh + 1) * 128]


def _mm_next_body(acc_ref, b_ref, w_ref, asd_ref, xw_ref, a_ref, *, h_in, heads):
    hcat = jnp.concatenate([acc_ref[hh] for hh in range(h_in)], axis=1)
    hb = hcat + b_ref[...]
    hb = jnp.where(hb > 0, hb, jnp.exp(hb) - 1.0)
    xwb = jnp.dot(hb, w_ref[...], preferred_element_type=jnp.float32)
    a_ref[...] = jnp.dot(xwb, asd_ref[...], preferred_element_type=jnp.float32)
    for hh in range(heads):
        xw_ref[hh] = xwb[:, hh * 128:(hh + 1) * 128]


def _mm_first(h, w, asd, heads):
    fin = h.shape[1]
    return pl.pallas_call(
        functools.partial(_mm_first_body, heads=heads),
        grid=(NB,),
        in_specs=[
            pl.BlockSpec((128, fin), lambda i: (i, 0)),
            pl.BlockSpec((fin, heads * 128), lambda i: (0, 0)),
            pl.BlockSpec((heads * 128, 16), lambda i: (0, 0)),
        ],
        out_specs=[
            pl.BlockSpec((heads, 128, 128), lambda i: (0, i, 0)),
            pl.BlockSpec((128, 16), lambda i: (i, 0)),
        ],
        out_shape=[
            jax.ShapeDtypeStruct((heads, NP, 128), jnp.float32),
            jax.ShapeDtypeStruct((NP, 16), jnp.float32),
        ],
    )(h, w, asd)


def _mm_next(acc, b, w, asd, h_in, heads):
    fin = h_in * 128
    return pl.pallas_call(
        functools.partial(_mm_next_body, h_in=h_in, heads=heads),
        grid=(NB,),
        in_specs=[
            pl.BlockSpec((h_in, 128, 128), lambda i: (0, i, 0)),
            pl.BlockSpec((1, fin), lambda i: (0, 0)),
            pl.BlockSpec((fin, heads * 128), lambda i: (0, 0)),
            pl.BlockSpec((heads * 128, 16), lambda i: (0, 0)),
        ],
        out_specs=[
            pl.BlockSpec((heads, 128, 128), lambda i: (0, i, 0)),
            pl.BlockSpec((128, 16), lambda i: (i, 0)),
        ],
        out_shape=[
            jax.ShapeDtypeStruct((heads, NP, 128), jnp.float32),
            jax.ShapeDtypeStruct((NP, 16), jnp.float32),
        ],
    )(acc, b, w, asd)


def _gat_edges_sc(heads, xw_flat, a_t, src3d, dst3d):
    mesh = plsc.VectorSubcoreMesh(core_axis_name="c", subcore_axis_name="s")

    @functools.partial(
        pl.kernel,
        mesh=mesh,
        compiler_params=pltpu.CompilerParams(
            use_tc_tiling_on_sc=False, needs_layout_passes=False),
        out_type=jax.ShapeDtypeStruct((heads, NP, 128), jnp.float32),
        scratch_types=[
            pltpu.VMEM((NP,), jnp.float32),
            pltpu.VMEM((NP,), jnp.float32),
            pltpu.VMEM((128,), jnp.int32),
            pltpu.VMEM((128,), jnp.int32),
            pltpu.VMEM((128,), jnp.int32),
            pltpu.VMEM((128,), jnp.float32),
            pltpu.VMEM((128, 128), jnp.float32),
            pltpu.VMEM((640,), jnp.float32),
            pltpu.VMEM((STRIPE,), jnp.float32),
            pltpu.VMEM_SHARED((NP, 128), jnp.float32),
            pltpu.VMEM_SHARED((NP,), jnp.float32),
            pltpu.SemaphoreType.DMA,
        ],
    )
    def k(xw_hbm, at_hbm, src_hbm, dst_hbm, acc_hbm,
          as_col, ad_col, sidx, sidx2, didx, exb, rows, denb, zcol,
          acc_s, den_s, gsem):
        c = lax.axis_index("c")
        s = lax.axis_index("s")
        zero16 = jnp.zeros((16,), jnp.float32)

        def zcol_body(i, _):
            zcol[pl.ds(16 * i, 16)] = zero16
            return 0
        lax.fori_loop(0, STRIPE // 16, zcol_body, 0)

        def head_body(h, carry):
            @pl.when((h % 2) == c)
            def _head():
                pltpu.sync_copy(at_hbm.at[h], as_col)
                pltpu.sync_copy(at_hbm.at[8 + h], ad_col)

                def zrow_body(r, _):
                    for g in range(8):
                        rows[r, pl.ds(16 * g, 16)] = zero16
                    return 0
                lax.fori_loop(0, 128, zrow_body, 0)
                pltpu.sync_copy(zcol, den_s.at[pl.ds(s * STRIPE, STRIPE)])
                base = s * STRIPE
                for t in range(4):
                    pltpu.sync_copy(rows,
                                    acc_s.at[pl.ds(base + t * 128, 128)])
                pltpu.sync_copy(rows.at[pl.ds(0, STRIPE - 512)],
                                acc_s.at[pl.ds(base + 512, STRIPE - 512)])
                plsc.subcore_barrier()

                def b_body(j, _):
                    pltpu.sync_copy(src_hbm.at[s, j], sidx)
                    pltpu.sync_copy(dst_hbm.at[s, j], didx)
                    for g in range(8):
                        si = sidx[pl.ds(16 * g, 16)]
                        di = didx[pl.ds(16 * g, 16)]
                        av = plsc.load_gather(as_col, [si])
                        dv = plsc.load_gather(ad_col, [di])
                        al = av + dv
                        al = jnp.where(al >= 0, al, 0.2 * al)
                        exb[pl.ds(16 * g, 16)] = jnp.exp(al)
                    pltpu.sync_copy(exb, den_s.at[didx], add=True)
                    return 0
                lax.fori_loop(0, NCH, b_body, 0)
                plsc.subcore_barrier()

                def c_body(j, _):
                    pltpu.sync_copy(src_hbm.at[s, j], sidx)
                    pltpu.sync_copy(dst_hbm.at[s, j], didx)
                    for g in range(8):
                        sidx2[pl.ds(16 * g, 16)] = (
                            sidx[pl.ds(16 * g, 16)] + (h * NP))
                    cp = pltpu.async_copy(xw_hbm.at[sidx2], rows, gsem)
                    for g in range(8):
                        si = sidx[pl.ds(16 * g, 16)]
                        di = didx[pl.ds(16 * g, 16)]
                        av = plsc.load_gather(as_col, [si])
                        dv = plsc.load_gather(ad_col, [di])
                        al = av + dv
                        al = jnp.where(al >= 0, al, 0.2 * al)
                        exb[pl.ds(16 * g, 16)] = jnp.exp(al)
                    cp.wait()

                    def r_body(q, _):
                        cfv = exb[pl.ds(16 * q, 16)]
                        for l in range(16):
                            cf = cfv[l]
                            r = 16 * q + l
                            for g in range(8):
                                rows[r, pl.ds(16 * g, 16)] = (
                                    rows[r, pl.ds(16 * g, 16)] * cf)
                        return 0
                    lax.fori_loop(0, 8, r_body, 0)
                    pltpu.sync_copy(rows, acc_s.at[didx], add=True)
                    return 0
                lax.fori_loop(0, NCH, c_body, 0)
                plsc.subcore_barrier()

                pltpu.sync_copy(den_s.at[pl.ds(base, STRIPE)],
                                denb.at[pl.ds(0, STRIPE)])
                for t in range(5):
                    cnt = 128 if t < 4 else STRIPE - 512
                    pltpu.sync_copy(
                        acc_s.at[pl.ds(base + t * 128, cnt)],
                        rows.at[pl.ds(0, cnt)])

                    def n_body(q, _):
                        dv = denb[pl.ds(t * 128 + 16 * q, 16)]
                        iv = 1.0 / (dv + 1e-16)
                        for l in range(16):
                            cf = iv[l]
                            for g in range(8):
                                rows[16 * q + l, pl.ds(16 * g, 16)] = (
                                    rows[16 * q + l, pl.ds(16 * g, 16)] * cf)
                        return 0
                    lax.fori_loop(0, 8, n_body, 0)
                    pltpu.sync_copy(
                        rows.at[pl.ds(0, cnt)],
                        acc_hbm.at[h, pl.ds(base + t * 128, cnt)])
                plsc.subcore_barrier()
            return carry

        lax.fori_loop(0, heads, head_body, 0)

    return k(xw_flat, a_t, src3d, dst3d)


def _pool_body(acc_ref, batch_ref, sums_ref, cnt_ref):
    i = pl.program_id(0)

    @pl.when(i == 0)
    def _init():
        sums_ref[...] = jnp.zeros_like(sums_ref)
        cnt_ref[...] = jnp.zeros_like(cnt_ref)

    b = batch_ref[0]
    gi = lax.broadcasted_iota(jnp.int32, (128, 128), 0)
    oh = (gi == b).astype(jnp.float32)
    hcat = jnp.concatenate([acc_ref[hh] for hh in range(6)], axis=1)
    sums_ref[...] += jnp.dot(oh, hcat, preferred_element_type=jnp.float32)
    cnt_ref[...] += jnp.sum(oh, axis=1, keepdims=True)


def _fin_body(sums_ref, cnt_ref, b3m_ref, wp_ref, bp_ref, o_ref):
    cnt = jnp.maximum(cnt_ref[...][:, 0:1], 1.0)
    pm = sums_ref[...] / cnt
    mh = (pm[:, 0:128] + pm[:, 128:256] + pm[:, 256:384] + pm[:, 384:512]
          + pm[:, 512:640] + pm[:, 640:768]) * (1.0 / 6.0)
    mh = mh + b3m_ref[...]
    o_ref[...] = jnp.dot(mh, wp_ref[...],
                         preferred_element_type=jnp.float32) + bp_ref[...]


def _pool(acc3, batch2d, b3m, wp_pad, bp):
    sums, cnt = pl.pallas_call(
        _pool_body,
        grid=(NB,),
        in_specs=[
            pl.BlockSpec((6, 128, 128), lambda i: (0, i, 0)),
            pl.BlockSpec((1, 1, 128), lambda i: (i, 0, 0)),
        ],
        out_specs=[
            pl.BlockSpec((128, 768), lambda i: (0, 0)),
            pl.BlockSpec((128, 128), lambda i: (0, 0)),
        ],
        out_shape=[
            jax.ShapeDtypeStruct((128, 768), jnp.float32),
            jax.ShapeDtypeStruct((128, 128), jnp.float32),
        ],
    )(acc3, batch2d)
    return pl.pallas_call(
        _fin_body,
        out_shape=jax.ShapeDtypeStruct((G, 128), jnp.float32),
    )(sums, cnt, b3m[None, :], wp_pad, bp[None, :])


def _att_mat(a_s, a_d, heads, fin):
    asd = jnp.zeros((fin, 16), jnp.float32)
    for h in range(heads):
        asd = asd.at[h * 128:(h + 1) * 128, h].set(a_s[0, h])
        asd = asd.at[h * 128:(h + 1) * 128, 8 + h].set(a_d[0, h])
    return asd


def kernel(x, edge_index, node_depth, batch, type_emb, attr_emb, depth_emb,
           W1, as1, ad1, b1, W2, as2, ad2, b2, W3, as3, ad3, b3, Wp, bp):
    f32 = jnp.float32
    h0 = (type_emb[x[:, 0]] + attr_emb[x[:, 1]]
          + depth_emb[node_depth.reshape(-1)])
    h0 = jnp.concatenate([h0, jnp.zeros((NP - N, 128), f32)], axis=0)

    loop = jnp.arange(N, dtype=jnp.int32)
    pad = jnp.full((ET - E - N,), SENT, jnp.int32)
    src3d = jnp.concatenate([edge_index[0], loop, pad]).reshape(16, NCH, 128)
    dst3d = jnp.concatenate([edge_index[1], loop, pad]).reshape(16, NCH, 128)

    W3p = jnp.pad(W3.reshape(512, 6, 121),
                  ((0, 0), (0, 0), (0, 7))).reshape(512, 768)
    as3p = jnp.pad(as3[0], ((0, 0), (0, 7)))[None]
    ad3p = jnp.pad(ad3[0], ((0, 0), (0, 7)))[None]
    b3m = jnp.pad(b3.reshape(6, 121), ((0, 0), (0, 7))).mean(axis=0)
    wp_pad = jnp.pad(Wp, ((0, 7), (0, 0)))

    asd1 = _att_mat(as1, ad1, 4, 512)
    asd2 = _att_mat(as2, ad2, 4, 512)
    asd3 = _att_mat(as3p, ad3p, 6, 768)

    xw, a = _mm_first(h0, W1, asd1, 4)
    acc = _gat_edges_sc(4, xw.reshape(4 * NP, 128), a.T, src3d, dst3d)
    xw, a = _mm_next(acc, b1[None, :], W2, asd2, 4, 4)
    acc = _gat_edges_sc(4, xw.reshape(4 * NP, 128), a.T, src3d, dst3d)
    xw, a = _mm_next(acc, b2[None, :], W3p, asd3, 4, 6)
    acc = _gat_edges_sc(6, xw.reshape(6 * NP, 128), a.T, src3d, dst3d)

    batch2d = jnp.concatenate(
        [batch.astype(jnp.int32), jnp.full((NP - N,), G + 7, jnp.int32)]
    ).reshape(NB, 1, 128)
    return _pool(acc, batch2d, b3m, wp_pad, bp)


# speedup vs baseline: 26.2004x; 1.6250x over previous
import functools

import jax
import jax.numpy as jnp
from jax import lax
from jax.experimental import pallas as pl
from jax.experimental.pallas import tpu as pltpu
from jax.experimental.pallas import tpu_sc as plsc

N = 10000
E = 320000
G = 128
NP = 10112
SENT = 10000
ET = 331776
PER_TILE = ET // 16
NCH = 162
NCH2 = 324
NB = NP // 128
STRIPE = NP // 16


def _mm_first_body(h_ref, w_ref, asd_ref, xw_ref, a_ref, *, heads):
    xwb = jnp.dot(h_ref[...], w_ref[...], preferred_element_type=jnp.float32)
    a_ref[...] = jnp.dot(xwb, asd_ref[...], preferred_element_type=jnp.float32)
    for hh in range(heads):
        xw_ref[hh] = xwb[:, hh * 128:(hh + 1) * 128]


def _mm_next_body(acc_ref, b_ref, w_ref, asd_ref, xw_ref, a_ref, *, h_in, heads):
    hcat = jnp.concatenate([acc_ref[hh] for hh in range(h_in)], axis=1)
    hb = hcat + b_ref[...]
    hb = jnp.where(hb > 0, hb, jnp.exp(hb) - 1.0)
    xwb = jnp.dot(hb, w_ref[...], preferred_element_type=jnp.float32)
    a_ref[...] = jnp.dot(xwb, asd_ref[...], preferred_element_type=jnp.float32)
    for hh in range(heads):
        xw_ref[hh] = xwb[:, hh * 128:(hh + 1) * 128]


def _mm_first(h, w, asd, heads):
    fin = h.shape[1]
    return pl.pallas_call(
        functools.partial(_mm_first_body, heads=heads),
        grid=(NB,),
        in_specs=[
            pl.BlockSpec((128, fin), lambda i: (i, 0)),
            pl.BlockSpec((fin, heads * 128), lambda i: (0, 0)),
            pl.BlockSpec((heads * 128, 16), lambda i: (0, 0)),
        ],
        out_specs=[
            pl.BlockSpec((heads, 128, 128), lambda i: (0, i, 0)),
            pl.BlockSpec((128, 16), lambda i: (i, 0)),
        ],
        out_shape=[
            jax.ShapeDtypeStruct((heads, NP, 128), jnp.float32),
            jax.ShapeDtypeStruct((NP, 16), jnp.float32),
        ],
    )(h, w, asd)


def _mm_next(acc, b, w, asd, h_in, heads):
    fin = h_in * 128
    return pl.pallas_call(
        functools.partial(_mm_next_body, h_in=h_in, heads=heads),
        grid=(NB,),
        in_specs=[
            pl.BlockSpec((h_in, 128, 128), lambda i: (0, i, 0)),
            pl.BlockSpec((1, fin), lambda i: (0, 0)),
            pl.BlockSpec((fin, heads * 128), lambda i: (0, 0)),
            pl.BlockSpec((heads * 128, 16), lambda i: (0, 0)),
        ],
        out_specs=[
            pl.BlockSpec((heads, 128, 128), lambda i: (0, i, 0)),
            pl.BlockSpec((128, 16), lambda i: (i, 0)),
        ],
        out_shape=[
            jax.ShapeDtypeStruct((heads, NP, 128), jnp.float32),
            jax.ShapeDtypeStruct((NP, 16), jnp.float32),
        ],
    )(acc, b, w, asd)


def _gat_edges_sc(heads, xw_flat, a_t, sd4d):
    mesh = plsc.VectorSubcoreMesh(core_axis_name="c", subcore_axis_name="s")

    @functools.partial(
        pl.kernel,
        mesh=mesh,
        compiler_params=pltpu.CompilerParams(
            use_tc_tiling_on_sc=False, needs_layout_passes=False),
        out_type=jax.ShapeDtypeStruct((heads, NP, 128), jnp.float32),
        scratch_types=[
            pltpu.VMEM((NP,), jnp.float32),
            pltpu.VMEM((NP,), jnp.float32),
            pltpu.VMEM((2, 64), jnp.int32),
            pltpu.VMEM((2, 64), jnp.int32),
            pltpu.VMEM((64,), jnp.int32),
            pltpu.VMEM((64,), jnp.int32),
            pltpu.VMEM((64,), jnp.float32),
            pltpu.VMEM((64, 128), jnp.float32),
            pltpu.VMEM((64, 128), jnp.float32),
            pltpu.VMEM((640,), jnp.float32),
            pltpu.VMEM((STRIPE,), jnp.float32),
            pltpu.VMEM_SHARED((NP, 128), jnp.float32),
            pltpu.VMEM_SHARED((NP,), jnp.float32),
            pltpu.SemaphoreType.DMA,
            pltpu.SemaphoreType.DMA,
            pltpu.SemaphoreType.DMA,
            pltpu.SemaphoreType.DMA,
        ],
    )
    def k(xw_hbm, at_hbm, sd_hbm, acc_hbm,
          as_col, ad_col, sd0, sd1, s20, s21, exb, rows0, rows1, denb, zcol,
          acc_s, den_s, gsem0, gsem1, ssem0, ssem1):
        c = lax.axis_index("c")
        s = lax.axis_index("s")
        zero16 = jnp.zeros((16,), jnp.float32)
        sds = (sd0, sd1)
        s2s = (s20, s21)
        rowss = (rows0, rows1)
        gsems = (gsem0, gsem1)
        ssems = (ssem0, ssem1)

        def zcol_body(i, _):
            zcol[pl.ds(16 * i, 16)] = zero16
            return 0
        lax.fori_loop(0, STRIPE // 16, zcol_body, 0)

        def load_chunk(h, j, p):
            pltpu.sync_copy(sd_hbm.at[s, j], sds[p])
            for g in range(4):
                s2s[p][pl.ds(16 * g, 16)] = (
                    sds[p][0, pl.ds(16 * g, 16)] + h * NP)
            return pltpu.async_copy(xw_hbm.at[s2s[p]], rowss[p], gsems[p])

        def wait_scatter(p):
            pltpu.make_async_copy(
                rowss[p], acc_s.at[sds[p].at[1]], ssems[p]).wait()

        def do_chunk(h, p):
            for g in range(4):
                si = sds[p][0, pl.ds(16 * g, 16)]
                di = sds[p][1, pl.ds(16 * g, 16)]
                av = plsc.load_gather(as_col, [si])
                dv = plsc.load_gather(ad_col, [di])
                al = av + dv
                al = jnp.where(al >= 0, al, 0.2 * al)
                exb[pl.ds(16 * g, 16)] = jnp.exp(al)
            pltpu.sync_copy(exb, den_s.at[sds[p].at[1]], add=True)
            pltpu.make_async_copy(
                xw_hbm.at[s2s[p]], rowss[p], gsems[p]).wait()

            def r_body(q, _):
                cfv = exb[pl.ds(16 * q, 16)]
                for l in range(16):
                    cf = cfv[l]
                    r = 16 * q + l
                    for g in range(8):
                        rowss[p][r, pl.ds(16 * g, 16)] = (
                            rowss[p][r, pl.ds(16 * g, 16)] * cf)
                return 0
            lax.fori_loop(0, 4, r_body, 0)
            pltpu.async_copy(rowss[p], acc_s.at[sds[p].at[1]], ssems[p],
                             add=True)

        def head_body(h, carry):
            @pl.when((h % 2) == c)
            def _head():
                pltpu.sync_copy(at_hbm.at[h], as_col)
                pltpu.sync_copy(at_hbm.at[8 + h], ad_col)

                def zrow_body(r, _):
                    for g in range(8):
                        rows0[r, pl.ds(16 * g, 16)] = zero16
                    return 0
                lax.fori_loop(0, 64, zrow_body, 0)
                pltpu.sync_copy(zcol, den_s.at[pl.ds(s * STRIPE, STRIPE)])
                base = s * STRIPE
                for t in range(9):
                    pltpu.sync_copy(rows0,
                                    acc_s.at[pl.ds(base + t * 64, 64)])
                pltpu.sync_copy(rows0.at[pl.ds(0, STRIPE - 576)],
                                acc_s.at[pl.ds(base + 576, STRIPE - 576)])
                plsc.subcore_barrier()

                load_chunk(h, 0, 0)

                def pair_body(i, _):
                    @pl.when(i > 0)
                    def _():
                        wait_scatter(1)
                    load_chunk(h, 2 * i + 1, 1)
                    do_chunk(h, 0)
                    wait_scatter(0)

                    @pl.when(i < (NCH2 // 2 - 1))
                    def _():
                        load_chunk(h, 2 * i + 2, 0)
                    do_chunk(h, 1)
                    return 0
                lax.fori_loop(0, NCH2 // 2, pair_body, 0)
                wait_scatter(1)
                plsc.subcore_barrier()

                pltpu.sync_copy(den_s.at[pl.ds(base, STRIPE)],
                                denb.at[pl.ds(0, STRIPE)])
                for t in range(10):
                    cnt = 64 if t < 9 else STRIPE - 576
                    pltpu.sync_copy(
                        acc_s.at[pl.ds(base + t * 64, cnt)],
                        rows0.at[pl.ds(0, cnt)])

                    def n_body(q, _):
                        dv = denb[pl.ds(t * 64 + 16 * q, 16)]
                        iv = 1.0 / (dv + 1e-16)
                        for l in range(16):
                            cf = iv[l]
                            for g in range(8):
                                rows0[16 * q + l, pl.ds(16 * g, 16)] = (
                                    rows0[16 * q + l, pl.ds(16 * g, 16)] * cf)
                        return 0
                    lax.fori_loop(0, 4, n_body, 0)
                    pltpu.sync_copy(
                        rows0.at[pl.ds(0, cnt)],
                        acc_hbm.at[h, pl.ds(base + t * 64, cnt)])
                plsc.subcore_barrier()
            return carry

        lax.fori_loop(0, heads, head_body, 0)

    return k(xw_flat, a_t, sd4d)


def _pool_body(acc_ref, batch_ref, sums_ref, cnt_ref):
    i = pl.program_id(0)

    @pl.when(i == 0)
    def _init():
        sums_ref[...] = jnp.zeros_like(sums_ref)
        cnt_ref[...] = jnp.zeros_like(cnt_ref)

    b = batch_ref[0]
    gi = lax.broadcasted_iota(jnp.int32, (128, 128), 0)
    oh = (gi == b).astype(jnp.float32)
    hcat = jnp.concatenate([acc_ref[hh] for hh in range(6)], axis=1)
    sums_ref[...] += jnp.dot(oh, hcat, preferred_element_type=jnp.float32)
    cnt_ref[...] += jnp.sum(oh, axis=1, keepdims=True)


def _fin_body(sums_ref, cnt_ref, b3m_ref, wp_ref, bp_ref, o_ref):
    cnt = jnp.maximum(cnt_ref[...][:, 0:1], 1.0)
    pm = sums_ref[...] / cnt
    mh = (pm[:, 0:128] + pm[:, 128:256] + pm[:, 256:384] + pm[:, 384:512]
          + pm[:, 512:640] + pm[:, 640:768]) * (1.0 / 6.0)
    mh = mh + b3m_ref[...]
    o_ref[...] = jnp.dot(mh, wp_ref[...],
                         preferred_element_type=jnp.float32) + bp_ref[...]


def _pool(acc3, batch2d, b3m, wp_pad, bp):
    sums, cnt = pl.pallas_call(
        _pool_body,
        grid=(NB,),
        in_specs=[
            pl.BlockSpec((6, 128, 128), lambda i: (0, i, 0)),
            pl.BlockSpec((1, 1, 128), lambda i: (i, 0, 0)),
        ],
        out_specs=[
            pl.BlockSpec((128, 768), lambda i: (0, 0)),
            pl.BlockSpec((128, 128), lambda i: (0, 0)),
        ],
        out_shape=[
            jax.ShapeDtypeStruct((128, 768), jnp.float32),
            jax.ShapeDtypeStruct((128, 128), jnp.float32),
        ],
    )(acc3, batch2d)
    return pl.pallas_call(
        _fin_body,
        out_shape=jax.ShapeDtypeStruct((G, 128), jnp.float32),
    )(sums, cnt, b3m[None, :], wp_pad, bp[None, :])


def _att_mat(a_s, a_d, heads, fin):
    asd = jnp.zeros((fin, 16), jnp.float32)
    for h in range(heads):
        asd = asd.at[h * 128:(h + 1) * 128, h].set(a_s[0, h])
        asd = asd.at[h * 128:(h + 1) * 128, 8 + h].set(a_d[0, h])
    return asd


def kernel(x, edge_index, node_depth, batch, type_emb, attr_emb, depth_emb,
           W1, as1, ad1, b1, W2, as2, ad2, b2, W3, as3, ad3, b3, Wp, bp):
    f32 = jnp.float32
    h0 = (type_emb[x[:, 0]] + attr_emb[x[:, 1]]
          + depth_emb[node_depth.reshape(-1)])
    h0 = jnp.concatenate([h0, jnp.zeros((NP - N, 128), f32)], axis=0)

    loop = jnp.arange(N, dtype=jnp.int32)
    pad = jnp.full((ET - E - N,), SENT, jnp.int32)
    srcp = jnp.concatenate([edge_index[0], loop, pad]).reshape(16, NCH2, 64)
    dstp = jnp.concatenate([edge_index[1], loop, pad]).reshape(16, NCH2, 64)
    sd4d = jnp.stack([srcp, dstp], axis=2)

    W3p = jnp.pad(W3.reshape(512, 6, 121),
                  ((0, 0), (0, 0), (0, 7))).reshape(512, 768)
    as3p = jnp.pad(as3[0], ((0, 0), (0, 7)))[None]
    ad3p = jnp.pad(ad3[0], ((0, 0), (0, 7)))[None]
    b3m = jnp.pad(b3.reshape(6, 121), ((0, 0), (0, 7))).mean(axis=0)
    wp_pad = jnp.pad(Wp, ((0, 7), (0, 0)))

    asd1 = _att_mat(as1, ad1, 4, 512)
    asd2 = _att_mat(as2, ad2, 4, 512)
    asd3 = _att_mat(as3p, ad3p, 6, 768)

    xw, a = _mm_first(h0, W1, asd1, 4)
    acc = _gat_edges_sc(4, xw.reshape(4 * NP, 128), a.T, sd4d)
    xw, a = _mm_next(acc, b1[None, :], W2, asd2, 4, 4)
    acc = _gat_edges_sc(4, xw.reshape(4 * NP, 128), a.T, sd4d)
    xw, a = _mm_next(acc, b2[None, :], W3p, asd3, 4, 6)
    acc = _gat_edges_sc(6, xw.reshape(6 * NP, 128), a.T, sd4d)

    batch2d = jnp.concatenate(
        [batch.astype(jnp.int32), jnp.full((NP - N,), G + 7, jnp.int32)]
    ).reshape(NB, 1, 128)
    return _pool(acc, batch2d, b3m, wp_pad, bp)


# speedup vs baseline: 27.5562x; 1.0517x over previous
import functools

import jax
import jax.numpy as jnp
from jax import lax
from jax.experimental import pallas as pl
from jax.experimental.pallas import tpu as pltpu
from jax.experimental.pallas import tpu_sc as plsc

N = 10000
E = 320000
G = 128
NP = 10112
SENT = 10000
ET = 331776
PER_TILE = ET // 16
NCH = 162
NCH2 = 324
NB = NP // 128
STRIPE = NP // 16


def _mm_first_body(h_ref, w_ref, asd_ref, xw_ref, a_ref, *, heads):
    xwb = jnp.dot(h_ref[...], w_ref[...], preferred_element_type=jnp.float32)
    a_ref[...] = jnp.dot(xwb, asd_ref[...], preferred_element_type=jnp.float32)
    for hh in range(heads):
        xw_ref[hh] = xwb[:, hh * 128:(hh + 1) * 128]


def _mm_next_body(acc_ref, b_ref, w_ref, asd_ref, xw_ref, a_ref, *, h_in, heads):
    hcat = jnp.concatenate([acc_ref[hh] for hh in range(h_in)], axis=1)
    hb = hcat + b_ref[...]
    hb = jnp.where(hb > 0, hb, jnp.exp(hb) - 1.0)
    xwb = jnp.dot(hb, w_ref[...], preferred_element_type=jnp.float32)
    a_ref[...] = jnp.dot(xwb, asd_ref[...], preferred_element_type=jnp.float32)
    for hh in range(heads):
        xw_ref[hh] = xwb[:, hh * 128:(hh + 1) * 128]


def _mm_first(h, w, asd, heads):
    fin = h.shape[1]
    return pl.pallas_call(
        functools.partial(_mm_first_body, heads=heads),
        grid=(NB,),
        in_specs=[
            pl.BlockSpec((128, fin), lambda i: (i, 0)),
            pl.BlockSpec((fin, heads * 128), lambda i: (0, 0)),
            pl.BlockSpec((heads * 128, 16), lambda i: (0, 0)),
        ],
        out_specs=[
            pl.BlockSpec((heads, 128, 128), lambda i: (0, i, 0)),
            pl.BlockSpec((128, 16), lambda i: (i, 0)),
        ],
        out_shape=[
            jax.ShapeDtypeStruct((heads, NP, 128), jnp.float32),
            jax.ShapeDtypeStruct((NP, 16), jnp.float32),
        ],
    )(h, w, asd)


def _mm_next(acc, b, w, asd, h_in, heads):
    fin = h_in * 128
    return pl.pallas_call(
        functools.partial(_mm_next_body, h_in=h_in, heads=heads),
        grid=(NB,),
        in_specs=[
            pl.BlockSpec((h_in, 128, 128), lambda i: (0, i, 0)),
            pl.BlockSpec((1, fin), lambda i: (0, 0)),
            pl.BlockSpec((fin, heads * 128), lambda i: (0, 0)),
            pl.BlockSpec((heads * 128, 16), lambda i: (0, 0)),
        ],
        out_specs=[
            pl.BlockSpec((heads, 128, 128), lambda i: (0, i, 0)),
            pl.BlockSpec((128, 16), lambda i: (i, 0)),
        ],
        out_shape=[
            jax.ShapeDtypeStruct((heads, NP, 128), jnp.float32),
            jax.ShapeDtypeStruct((NP, 16), jnp.float32),
        ],
    )(acc, b, w, asd)


def _gat_edges_sc(heads, xw_flat, a_t, sd4d):
    mesh = plsc.VectorSubcoreMesh(core_axis_name="c", subcore_axis_name="s")

    @functools.partial(
        pl.kernel,
        mesh=mesh,
        compiler_params=pltpu.CompilerParams(
            use_tc_tiling_on_sc=False, needs_layout_passes=False),
        out_type=jax.ShapeDtypeStruct((heads, NP, 128), jnp.float32),
        scratch_types=[
            pltpu.VMEM((NP,), jnp.float32),
            pltpu.VMEM((NP,), jnp.float32),
            pltpu.VMEM((2, 64), jnp.int32),
            pltpu.VMEM((2, 64), jnp.int32),
            pltpu.VMEM((64,), jnp.int32),
            pltpu.VMEM((64,), jnp.int32),
            pltpu.VMEM((64,), jnp.float32),
            pltpu.VMEM((64,), jnp.float32),
            pltpu.VMEM((64, 128), jnp.float32),
            pltpu.VMEM((64, 128), jnp.float32),
            pltpu.VMEM((640,), jnp.float32),
            pltpu.VMEM((STRIPE,), jnp.float32),
            pltpu.VMEM_SHARED((NP, 128), jnp.float32),
            pltpu.VMEM_SHARED((NP,), jnp.float32),
            pltpu.SemaphoreType.DMA,
            pltpu.SemaphoreType.DMA,
            pltpu.SemaphoreType.DMA,
            pltpu.SemaphoreType.DMA,
            pltpu.SemaphoreType.DMA,
            pltpu.SemaphoreType.DMA,
        ],
    )
    def k(xw_hbm, at_hbm, sd_hbm, acc_hbm,
          as_col, ad_col, sd0, sd1, s20, s21, exb0, exb1, rows0, rows1,
          denb, zcol, acc_s, den_s,
          gsem0, gsem1, ssem0, ssem1, dsem0, dsem1):
        c = lax.axis_index("c")
        s = lax.axis_index("s")
        zero16 = jnp.zeros((16,), jnp.float32)
        sds = (sd0, sd1)
        s2s = (s20, s21)
        exbs = (exb0, exb1)
        rowss = (rows0, rows1)
        gsems = (gsem0, gsem1)
        ssems = (ssem0, ssem1)
        dsems = (dsem0, dsem1)

        def zcol_body(i, _):
            zcol[pl.ds(16 * i, 16)] = zero16
            return 0
        lax.fori_loop(0, STRIPE // 16, zcol_body, 0)

        def load_chunk(h, j, p):
            pltpu.sync_copy(sd_hbm.at[s, j], sds[p])
            for g in range(4):
                s2s[p][pl.ds(16 * g, 16)] = (
                    sds[p][0, pl.ds(16 * g, 16)] + h * NP)
            return pltpu.async_copy(xw_hbm.at[s2s[p]], rowss[p], gsems[p])

        def wait_scatter(p):
            pltpu.make_async_copy(
                rowss[p], acc_s.at[sds[p].at[1]], ssems[p]).wait()

        def wait_den(p):
            pltpu.make_async_copy(
                exbs[p], den_s.at[sds[p].at[1]], dsems[p]).wait()

        def do_chunk(h, p):
            for g in range(4):
                si = sds[p][0, pl.ds(16 * g, 16)]
                di = sds[p][1, pl.ds(16 * g, 16)]
                av = plsc.load_gather(as_col, [si])
                dv = plsc.load_gather(ad_col, [di])
                al = av + dv
                al = jnp.where(al >= 0, al, 0.2 * al)
                exbs[p][pl.ds(16 * g, 16)] = jnp.exp(al)
            pltpu.async_copy(exbs[p], den_s.at[sds[p].at[1]], dsems[p],
                             add=True)
            pltpu.make_async_copy(
                xw_hbm.at[s2s[p]], rowss[p], gsems[p]).wait()

            def r_body(q, _):
                cfv = exbs[p][pl.ds(16 * q, 16)]
                for l in range(16):
                    cf = cfv[l]
                    r = 16 * q + l
                    for g in range(8):
                        rowss[p][r, pl.ds(16 * g, 16)] = (
                            rowss[p][r, pl.ds(16 * g, 16)] * cf)
                return 0
            lax.fori_loop(0, 4, r_body, 0)
            pltpu.async_copy(rowss[p], acc_s.at[sds[p].at[1]], ssems[p],
                             add=True)

        def head_body(h, carry):
            @pl.when((h % 2) == c)
            def _head():
                pltpu.sync_copy(at_hbm.at[h], as_col)
                pltpu.sync_copy(at_hbm.at[8 + h], ad_col)

                def zrow_body(r, _):
                    for g in range(8):
                        rows0[r, pl.ds(16 * g, 16)] = zero16
                    return 0
                lax.fori_loop(0, 64, zrow_body, 0)
                pltpu.sync_copy(zcol, den_s.at[pl.ds(s * STRIPE, STRIPE)])
                base = s * STRIPE
                for t in range(9):
                    pltpu.sync_copy(rows0,
                                    acc_s.at[pl.ds(base + t * 64, 64)])
                pltpu.sync_copy(rows0.at[pl.ds(0, STRIPE - 576)],
                                acc_s.at[pl.ds(base + 576, STRIPE - 576)])
                plsc.subcore_barrier()

                load_chunk(h, 0, 0)

                def pair_body(i, _):
                    @pl.when(i > 0)
                    def _():
                        wait_scatter(1)
                        wait_den(1)
                    load_chunk(h, 2 * i + 1, 1)
                    do_chunk(h, 0)
                    do_chunk(h, 1)
                    wait_scatter(0)
                    wait_den(0)

                    @pl.when(i < (NCH2 // 2 - 1))
                    def _():
                        load_chunk(h, 2 * i + 2, 0)
                    return 0
                lax.fori_loop(0, NCH2 // 2, pair_body, 0)
                wait_scatter(1)
                wait_den(1)
                plsc.subcore_barrier()

                pltpu.sync_copy(den_s.at[pl.ds(base, STRIPE)],
                                denb.at[pl.ds(0, STRIPE)])
                for t in range(10):
                    cnt = 64 if t < 9 else STRIPE - 576
                    pltpu.sync_copy(
                        acc_s.at[pl.ds(base + t * 64, cnt)],
                        rows0.at[pl.ds(0, cnt)])

                    def n_body(q, _):
                        dv = denb[pl.ds(t * 64 + 16 * q, 16)]
                        iv = 1.0 / (dv + 1e-16)
                        for l in range(16):
                            cf = iv[l]
                            for g in range(8):
                                rows0[16 * q + l, pl.ds(16 * g, 16)] = (
                                    rows0[16 * q + l, pl.ds(16 * g, 16)] * cf)
                        return 0
                    lax.fori_loop(0, 4, n_body, 0)
                    pltpu.sync_copy(
                        rows0.at[pl.ds(0, cnt)],
                        acc_hbm.at[h, pl.ds(base + t * 64, cnt)])
                plsc.subcore_barrier()
            return carry

        lax.fori_loop(0, heads, head_body, 0)

    return k(xw_flat, a_t, sd4d)


def _pool_body(acc_ref, batch_ref, sums_ref, cnt_ref):
    i = pl.program_id(0)

    @pl.when(i == 0)
    def _init():
        sums_ref[...] = jnp.zeros_like(sums_ref)
        cnt_ref[...] = jnp.zeros_like(cnt_ref)

    b = batch_ref[0]
    gi = lax.broadcasted_iota(jnp.int32, (128, 128), 0)
    oh = (gi == b).astype(jnp.float32)
    hcat = jnp.concatenate([acc_ref[hh] for hh in range(6)], axis=1)
    sums_ref[...] += jnp.dot(oh, hcat, preferred_element_type=jnp.float32)
    cnt_ref[...] += jnp.sum(oh, axis=1, keepdims=True)


def _fin_body(sums_ref, cnt_ref, b3m_ref, wp_ref, bp_ref, o_ref):
    cnt = jnp.maximum(cnt_ref[...][:, 0:1], 1.0)
    pm = sums_ref[...] / cnt
    mh = (pm[:, 0:128] + pm[:, 128:256] + pm[:, 256:384] + pm[:, 384:512]
          + pm[:, 512:640] + pm[:, 640:768]) * (1.0 / 6.0)
    mh = mh + b3m_ref[...]
    o_ref[...] = jnp.dot(mh, wp_ref[...],
                         preferred_element_type=jnp.float32) + bp_ref[...]


def _pool(acc3, batch2d, b3m, wp_pad, bp):
    sums, cnt = pl.pallas_call(
        _pool_body,
        grid=(NB,),
        in_specs=[
            pl.BlockSpec((6, 128, 128), lambda i: (0, i, 0)),
            pl.BlockSpec((1, 1, 128), lambda i: (i, 0, 0)),
        ],
        out_specs=[
            pl.BlockSpec((128, 768), lambda i: (0, 0)),
            pl.BlockSpec((128, 128), lambda i: (0, 0)),
        ],
        out_shape=[
            jax.ShapeDtypeStruct((128, 768), jnp.float32),
            jax.ShapeDtypeStruct((128, 128), jnp.float32),
        ],
    )(acc3, batch2d)
    return pl.pallas_call(
        _fin_body,
        out_shape=jax.ShapeDtypeStruct((G, 128), jnp.float32),
    )(sums, cnt, b3m[None, :], wp_pad, bp[None, :])


def _att_mat(a_s, a_d, heads, fin):
    asd = jnp.zeros((fin, 16), jnp.float32)
    for h in range(heads):
        asd = asd.at[h * 128:(h + 1) * 128, h].set(a_s[0, h])
        asd = asd.at[h * 128:(h + 1) * 128, 8 + h].set(a_d[0, h])
    return asd


def kernel(x, edge_index, node_depth, batch, type_emb, attr_emb, depth_emb,
           W1, as1, ad1, b1, W2, as2, ad2, b2, W3, as3, ad3, b3, Wp, bp):
    f32 = jnp.float32
    h0 = (type_emb[x[:, 0]] + attr_emb[x[:, 1]]
          + depth_emb[node_depth.reshape(-1)])
    h0 = jnp.concatenate([h0, jnp.zeros((NP - N, 128), f32)], axis=0)

    loop = jnp.arange(N, dtype=jnp.int32)
    pad = jnp.full((ET - E - N,), SENT, jnp.int32)
    srcp = jnp.concatenate([edge_index[0], loop, pad]).reshape(16, NCH2, 64)
    dstp = jnp.concatenate([edge_index[1], loop, pad]).reshape(16, NCH2, 64)
    sd4d = jnp.stack([srcp, dstp], axis=2)

    W3p = jnp.pad(W3.reshape(512, 6, 121),
                  ((0, 0), (0, 0), (0, 7))).reshape(512, 768)
    as3p = jnp.pad(as3[0], ((0, 0), (0, 7)))[None]
    ad3p = jnp.pad(ad3[0], ((0, 0), (0, 7)))[None]
    b3m = jnp.pad(b3.reshape(6, 121), ((0, 0), (0, 7))).mean(axis=0)
    wp_pad = jnp.pad(Wp, ((0, 7), (0, 0)))

    asd1 = _att_mat(as1, ad1, 4, 512)
    asd2 = _att_mat(as2, ad2, 4, 512)
    asd3 = _att_mat(as3p, ad3p, 6, 768)

    xw, a = _mm_first(h0, W1, asd1, 4)
    acc = _gat_edges_sc(4, xw.reshape(4 * NP, 128), a.T, sd4d)
    xw, a = _mm_next(acc, b1[None, :], W2, asd2, 4, 4)
    acc = _gat_edges_sc(4, xw.reshape(4 * NP, 128), a.T, sd4d)
    xw, a = _mm_next(acc, b2[None, :], W3p, asd3, 4, 6)
    acc = _gat_edges_sc(6, xw.reshape(6 * NP, 128), a.T, sd4d)

    batch2d = jnp.concatenate(
        [batch.astype(jnp.int32), jnp.full((NP - N,), G + 7, jnp.int32)]
    ).reshape(NB, 1, 128)
    return _pool(acc, batch2d, b3m, wp_pad, bp)


# speedup vs baseline: 32.7545x; 1.1886x over previous
import functools

import jax
import jax.numpy as jnp
from jax import lax
from jax.experimental import pallas as pl
from jax.experimental.pallas import tpu as pltpu
from jax.experimental.pallas import tpu_sc as plsc

N = 10000
E = 320000
G = 128
NP = 10112
SENT = 10000
ET = 331776
PER_TILE = ET // 16
NCH = 162
NCH2 = 324
NB = NP // 128
STRIPE = NP // 16


def _mm_first_body(h_ref, w_ref, asd_ref, xw_ref, a_ref, *, heads):
    xwb = jnp.dot(h_ref[...], w_ref[...], preferred_element_type=jnp.float32)
    a_ref[...] = jnp.dot(xwb, asd_ref[...], preferred_element_type=jnp.float32)
    for hh in range(heads):
        xw_ref[hh] = xwb[:, hh * 128:(hh + 1) * 128]


def _mm_next_body(acc_ref, b_ref, w_ref, asd_ref, xw_ref, a_ref, *, h_in, heads):
    hcat = jnp.concatenate([acc_ref[hh] for hh in range(h_in)], axis=1)
    hb = hcat + b_ref[...]
    hb = jnp.where(hb > 0, hb, jnp.exp(hb) - 1.0)
    xwb = jnp.dot(hb, w_ref[...], preferred_element_type=jnp.float32)
    a_ref[...] = jnp.dot(xwb, asd_ref[...], preferred_element_type=jnp.float32)
    for hh in range(heads):
        xw_ref[hh] = xwb[:, hh * 128:(hh + 1) * 128]


def _mm_first(h, w, asd, heads):
    fin = h.shape[1]
    return pl.pallas_call(
        functools.partial(_mm_first_body, heads=heads),
        grid=(NB,),
        in_specs=[
            pl.BlockSpec((128, fin), lambda i: (i, 0)),
            pl.BlockSpec((fin, heads * 128), lambda i: (0, 0)),
            pl.BlockSpec((heads * 128, 16), lambda i: (0, 0)),
        ],
        out_specs=[
            pl.BlockSpec((heads, 128, 128), lambda i: (0, i, 0)),
            pl.BlockSpec((128, 16), lambda i: (i, 0)),
        ],
        out_shape=[
            jax.ShapeDtypeStruct((heads, NP, 128), jnp.float32),
            jax.ShapeDtypeStruct((NP, 16), jnp.float32),
        ],
    )(h, w, asd)


def _mm_next(acc, b, w, asd, h_in, heads):
    fin = h_in * 128
    return pl.pallas_call(
        functools.partial(_mm_next_body, h_in=h_in, heads=heads),
        grid=(NB,),
        in_specs=[
            pl.BlockSpec((h_in, 128, 128), lambda i: (0, i, 0)),
            pl.BlockSpec((1, fin), lambda i: (0, 0)),
            pl.BlockSpec((fin, heads * 128), lambda i: (0, 0)),
            pl.BlockSpec((heads * 128, 16), lambda i: (0, 0)),
        ],
        out_specs=[
            pl.BlockSpec((heads, 128, 128), lambda i: (0, i, 0)),
            pl.BlockSpec((128, 16), lambda i: (i, 0)),
        ],
        out_shape=[
            jax.ShapeDtypeStruct((heads, NP, 128), jnp.float32),
            jax.ShapeDtypeStruct((NP, 16), jnp.float32),
        ],
    )(acc, b, w, asd)


def _gat_edges_sc(heads, xw_flat, a_t, sd4d):
    mesh = plsc.VectorSubcoreMesh(core_axis_name="c", subcore_axis_name="s")

    @functools.partial(
        pl.kernel,
        mesh=mesh,
        compiler_params=pltpu.CompilerParams(
            use_tc_tiling_on_sc=False, needs_layout_passes=False),
        out_type=jax.ShapeDtypeStruct((heads, NP, 128), jnp.float32),
        scratch_types=[
            pltpu.VMEM((NP,), jnp.float32),
            pltpu.VMEM((NP,), jnp.float32),
            pltpu.VMEM((2, 64), jnp.int32),
            pltpu.VMEM((2, 64), jnp.int32),
            pltpu.VMEM((2, 64), jnp.int32),
            pltpu.VMEM((64,), jnp.int32),
            pltpu.VMEM((64,), jnp.int32),
            pltpu.VMEM((64,), jnp.int32),
            pltpu.VMEM((64,), jnp.float32),
            pltpu.VMEM((64,), jnp.float32),
            pltpu.VMEM((64,), jnp.float32),
            pltpu.VMEM((64, 128), jnp.float32),
            pltpu.VMEM((64, 128), jnp.float32),
            pltpu.VMEM((64, 128), jnp.float32),
            pltpu.VMEM((640,), jnp.float32),
            pltpu.VMEM((STRIPE,), jnp.float32),
            pltpu.VMEM_SHARED((NP, 128), jnp.float32),
            pltpu.VMEM_SHARED((NP,), jnp.float32),
        ] + [pltpu.SemaphoreType.DMA] * 9,
    )
    def k(xw_hbm, at_hbm, sd_hbm, acc_hbm,
          as_col, ad_col, sd0, sd1, sd2, s20, s21, s22, exb0, exb1, exb2,
          rows0, rows1, rows2, denb, zcol, acc_s, den_s,
          gsem0, gsem1, gsem2, ssem0, ssem1, ssem2, dsem0, dsem1, dsem2):
        c = lax.axis_index("c")
        s = lax.axis_index("s")
        zero16 = jnp.zeros((16,), jnp.float32)
        sds = (sd0, sd1, sd2)
        s2s = (s20, s21, s22)
        exbs = (exb0, exb1, exb2)
        rowss = (rows0, rows1, rows2)
        gsems = (gsem0, gsem1, gsem2)
        ssems = (ssem0, ssem1, ssem2)
        dsems = (dsem0, dsem1, dsem2)

        def zcol_body(i, _):
            zcol[pl.ds(16 * i, 16)] = zero16
            return 0
        lax.fori_loop(0, STRIPE // 16, zcol_body, 0)

        def load_chunk(h, j, p):
            pltpu.sync_copy(sd_hbm.at[s, j], sds[p])
            for g in range(4):
                s2s[p][pl.ds(16 * g, 16)] = (
                    sds[p][0, pl.ds(16 * g, 16)] + h * NP)
            return pltpu.async_copy(xw_hbm.at[s2s[p]], rowss[p], gsems[p])

        def wait_scatter(p):
            pltpu.make_async_copy(
                rowss[p], acc_s.at[sds[p].at[1]], ssems[p]).wait()

        def wait_den(p):
            pltpu.make_async_copy(
                exbs[p], den_s.at[sds[p].at[1]], dsems[p]).wait()

        def do_chunk(h, p):
            for g in range(4):
                si = sds[p][0, pl.ds(16 * g, 16)]
                di = sds[p][1, pl.ds(16 * g, 16)]
                av = plsc.load_gather(as_col, [si])
                dv = plsc.load_gather(ad_col, [di])
                al = av + dv
                al = jnp.where(al >= 0, al, 0.2 * al)
                exbs[p][pl.ds(16 * g, 16)] = jnp.exp(al)
            pltpu.async_copy(exbs[p], den_s.at[sds[p].at[1]], dsems[p],
                             add=True)
            pltpu.make_async_copy(
                xw_hbm.at[s2s[p]], rowss[p], gsems[p]).wait()

            def r_body(q, _):
                cfv = exbs[p][pl.ds(16 * q, 16)]
                for l in range(16):
                    cf = cfv[l]
                    r = 16 * q + l
                    for g in range(8):
                        rowss[p][r, pl.ds(16 * g, 16)] = (
                            rowss[p][r, pl.ds(16 * g, 16)] * cf)
                return 0
            lax.fori_loop(0, 4, r_body, 0)
            pltpu.async_copy(rowss[p], acc_s.at[sds[p].at[1]], ssems[p],
                             add=True)

        def head_body(h, carry):
            @pl.when((h % 2) == c)
            def _head():
                pltpu.sync_copy(at_hbm.at[h], as_col)
                pltpu.sync_copy(at_hbm.at[8 + h], ad_col)

                def zrow_body(r, _):
                    for g in range(8):
                        rows0[r, pl.ds(16 * g, 16)] = zero16
                    return 0
                lax.fori_loop(0, 64, zrow_body, 0)
                pltpu.sync_copy(zcol, den_s.at[pl.ds(s * STRIPE, STRIPE)])
                base = s * STRIPE
                for t in range(9):
                    pltpu.sync_copy(rows0,
                                    acc_s.at[pl.ds(base + t * 64, 64)])
                pltpu.sync_copy(rows0.at[pl.ds(0, STRIPE - 576)],
                                acc_s.at[pl.ds(base + 576, STRIPE - 576)])
                plsc.subcore_barrier()

                load_chunk(h, 0, 0)
                load_chunk(h, 1, 1)

                def ch_body(j, _):
                    for p in range(3):
                        @pl.when((j % 3) == p)
                        def _(p=p):
                            do_chunk(h, p)

                            @pl.when(j >= 2)
                            def _():
                                wait_scatter((p + 1) % 3)
                                wait_den((p + 1) % 3)

                            @pl.when(j + 2 < NCH2)
                            def _():
                                load_chunk(h, j + 2, (p + 2) % 3)
                    return 0
                lax.fori_loop(0, NCH2, ch_body, 0)
                wait_scatter(1)
                wait_den(1)
                wait_scatter(2)
                wait_den(2)
                plsc.subcore_barrier()

                pltpu.sync_copy(den_s.at[pl.ds(base, STRIPE)],
                                denb.at[pl.ds(0, STRIPE)])
                for t in range(10):
                    cnt = 64 if t < 9 else STRIPE - 576
                    pltpu.sync_copy(
                        acc_s.at[pl.ds(base + t * 64, cnt)],
                        rows0.at[pl.ds(0, cnt)])

                    def n_body(q, _):
                        dv = denb[pl.ds(t * 64 + 16 * q, 16)]
                        iv = 1.0 / (dv + 1e-16)
                        for l in range(16):
                            cf = iv[l]
                            for g in range(8):
                                rows0[16 * q + l, pl.ds(16 * g, 16)] = (
                                    rows0[16 * q + l, pl.ds(16 * g, 16)] * cf)
                        return 0
                    lax.fori_loop(0, 4, n_body, 0)
                    pltpu.sync_copy(
                        rows0.at[pl.ds(0, cnt)],
                        acc_hbm.at[h, pl.ds(base + t * 64, cnt)])
                plsc.subcore_barrier()
            return carry

        lax.fori_loop(0, heads, head_body, 0)

    return k(xw_flat, a_t, sd4d)


def _pool_body(acc_ref, batch_ref, sums_ref, cnt_ref):
    i = pl.program_id(0)

    @pl.when(i == 0)
    def _init():
        sums_ref[...] = jnp.zeros_like(sums_ref)
        cnt_ref[...] = jnp.zeros_like(cnt_ref)

    b = batch_ref[0]
    gi = lax.broadcasted_iota(jnp.int32, (128, 128), 0)
    oh = (gi == b).astype(jnp.float32)
    hcat = jnp.concatenate([acc_ref[hh] for hh in range(6)], axis=1)
    sums_ref[...] += jnp.dot(oh, hcat, preferred_element_type=jnp.float32)
    cnt_ref[...] += jnp.sum(oh, axis=1, keepdims=True)


def _fin_body(sums_ref, cnt_ref, b3m_ref, wp_ref, bp_ref, o_ref):
    cnt = jnp.maximum(cnt_ref[...][:, 0:1], 1.0)
    pm = sums_ref[...] / cnt
    mh = (pm[:, 0:128] + pm[:, 128:256] + pm[:, 256:384] + pm[:, 384:512]
          + pm[:, 512:640] + pm[:, 640:768]) * (1.0 / 6.0)
    mh = mh + b3m_ref[...]
    o_ref[...] = jnp.dot(mh, wp_ref[...],
                         preferred_element_type=jnp.float32) + bp_ref[...]


def _pool(acc3, batch2d, b3m, wp_pad, bp):
    sums, cnt = pl.pallas_call(
        _pool_body,
        grid=(NB,),
        in_specs=[
            pl.BlockSpec((6, 128, 128), lambda i: (0, i, 0)),
            pl.BlockSpec((1, 1, 128), lambda i: (i, 0, 0)),
        ],
        out_specs=[
            pl.BlockSpec((128, 768), lambda i: (0, 0)),
            pl.BlockSpec((128, 128), lambda i: (0, 0)),
        ],
        out_shape=[
            jax.ShapeDtypeStruct((128, 768), jnp.float32),
            jax.ShapeDtypeStruct((128, 128), jnp.float32),
        ],
    )(acc3, batch2d)
    return pl.pallas_call(
        _fin_body,
        out_shape=jax.ShapeDtypeStruct((G, 128), jnp.float32),
    )(sums, cnt, b3m[None, :], wp_pad, bp[None, :])


def _att_mat(a_s, a_d, heads, fin):
    asd = jnp.zeros((fin, 16), jnp.float32)
    for h in range(heads):
        asd = asd.at[h * 128:(h + 1) * 128, h].set(a_s[0, h])
        asd = asd.at[h * 128:(h + 1) * 128, 8 + h].set(a_d[0, h])
    return asd


def kernel(x, edge_index, node_depth, batch, type_emb, attr_emb, depth_emb,
           W1, as1, ad1, b1, W2, as2, ad2, b2, W3, as3, ad3, b3, Wp, bp):
    f32 = jnp.float32
    h0 = (type_emb[x[:, 0]] + attr_emb[x[:, 1]]
          + depth_emb[node_depth.reshape(-1)])
    h0 = jnp.concatenate([h0, jnp.zeros((NP - N, 128), f32)], axis=0)

    loop = jnp.arange(N, dtype=jnp.int32)
    pad = jnp.full((ET - E - N,), SENT, jnp.int32)
    srcp = jnp.concatenate([edge_index[0], loop, pad]).reshape(16, NCH2, 64)
    dstp = jnp.concatenate([edge_index[1], loop, pad]).reshape(16, NCH2, 64)
    sd4d = jnp.stack([srcp, dstp], axis=2)

    W3p = jnp.pad(W3.reshape(512, 6, 121),
                  ((0, 0), (0, 0), (0, 7))).reshape(512, 768)
    as3p = jnp.pad(as3[0], ((0, 0), (0, 7)))[None]
    ad3p = jnp.pad(ad3[0], ((0, 0), (0, 7)))[None]
    b3m = jnp.pad(b3.reshape(6, 121), ((0, 0), (0, 7))).mean(axis=0)
    wp_pad = jnp.pad(Wp, ((0, 7), (0, 0)))

    asd1 = _att_mat(as1, ad1, 4, 512)
    asd2 = _att_mat(as2, ad2, 4, 512)
    asd3 = _att_mat(as3p, ad3p, 6, 768)

    xw, a = _mm_first(h0, W1, asd1, 4)
    acc = _gat_edges_sc(4, xw.reshape(4 * NP, 128), a.T, sd4d)
    xw, a = _mm_next(acc, b1[None, :], W2, asd2, 4, 4)
    acc = _gat_edges_sc(4, xw.reshape(4 * NP, 128), a.T, sd4d)
    xw, a = _mm_next(acc, b2[None, :], W3p, asd3, 4, 6)
    acc = _gat_edges_sc(6, xw.reshape(6 * NP, 128), a.T, sd4d)

    batch2d = jnp.concatenate(
        [batch.astype(jnp.int32), jnp.full((NP - N,), G + 7, jnp.int32)]
    ).reshape(NB, 1, 128)
    return _pool(acc, batch2d, b3m, wp_pad, bp)
